# Initial kernel scaffold; baseline (speedup 1.0000x reference)
import functools

import jax
import jax.numpy as jnp
from jax import lax
from jax.experimental import pallas as pl
from jax.experimental.pallas import tpu as pltpu
from jax.experimental.pallas import tpu_sc as plsc

N = 10000
NPAD = 10240
F0 = 128
H = 64
E = 320000
EPS = 1e-5
SLOPE = 0.01

NC = 2
NS = 16
NW = NC * NS
CHUNK = 128
NCH = -(-E // (NW * CHUNK))
EPW = NCH * CHUNK
EPAD = EPW * NW
ZPT = NPAD // NS
BR = 256
NBLK = NPAD // BR
BPW = 1024 // NW


def _sc_mesh():
    return plsc.VectorSubcoreMesh(
        core_axis_name="c", subcore_axis_name="s", num_cores=NC, num_subcores=NS
    )


def _deg_body(dst_hbm, out_hbm, dstv, onesv, zv, deg_sp):
    c = lax.axis_index("c")
    s = lax.axis_index("s")
    w = c * NS + s
    for k in range(CHUNK // 16):
        onesv[pl.ds(k * 16, 16)] = jnp.ones((16,), jnp.float32)

    def zbody(k, _):
        zv[pl.ds(k * 16, 16)] = jnp.zeros((16,), jnp.float32)
        return 0

    lax.fori_loop(0, ZPT // 16, zbody, 0)
    pltpu.sync_copy(zv, deg_sp.at[pl.ds(s * ZPT, ZPT)])
    pltpu.sync_copy(dst_hbm.at[w], dstv)
    plsc.subcore_barrier()

    def body(j, _):
        pltpu.sync_copy(
            onesv,
            deg_sp.at[plsc.Indices(dstv.at[j], ignored_value=-1)],
            add=True,
        )
        return 0

    lax.fori_loop(0, NCH, body, 0)
    plsc.subcore_barrier()
    pltpu.sync_copy(deg_sp.at[pl.ds(s * ZPT, ZPT)], out_hbm.at[c, pl.ds(s * ZPT, ZPT)])


def _deg_call(dstp):
    f = functools.partial(
        pl.kernel,
        out_type=jax.ShapeDtypeStruct((NC, NPAD), jnp.float32),
        mesh=_sc_mesh(),
        scratch_types=[
            pltpu.VMEM((NCH, CHUNK), jnp.int32),
            pltpu.VMEM((CHUNK,), jnp.float32),
            pltpu.VMEM((ZPT,), jnp.float32),
            pltpu.VMEM_SHARED((NPAD,), jnp.float32),
        ],
        name="gcn_deg",
    )(_deg_body)
    return f(dstp)


def _mp_body(y_hbm, src_hbm, dst_hbm, zero_hbm, out_hbm,
             srcv, dstv, rows, y_sp, acc_sp):
    c = lax.axis_index("c")
    s = lax.axis_index("s")
    w = c * NS + s
    r0 = s * ZPT
    pltpu.sync_copy(y_hbm.at[pl.ds(r0, ZPT)], y_sp.at[pl.ds(r0, ZPT)])
    pltpu.sync_copy(zero_hbm.at[pl.ds(r0, ZPT)], acc_sp.at[pl.ds(r0, ZPT)])
    pltpu.sync_copy(src_hbm.at[w], srcv)
    pltpu.sync_copy(dst_hbm.at[w], dstv)
    plsc.subcore_barrier()

    def body(j, _):
        pltpu.sync_copy(
            y_sp.at[plsc.Indices(srcv.at[j], ignored_value=-1)], rows
        )
        pltpu.sync_copy(
            rows,
            acc_sp.at[plsc.Indices(dstv.at[j], ignored_value=-1)],
            add=True,
        )
        return 0

    lax.fori_loop(0, NCH, body, 0)
    plsc.subcore_barrier()
    pltpu.sync_copy(acc_sp.at[pl.ds(r0, ZPT)], out_hbm.at[c, pl.ds(r0, ZPT)])


def _mp_call(y, srcp, dstp, zeros2d):
    f = functools.partial(
        pl.kernel,
        out_type=jax.ShapeDtypeStruct((NC, NPAD, H), jnp.float32),
        mesh=_sc_mesh(),
        scratch_types=[
            pltpu.VMEM((NCH, CHUNK), jnp.int32),
            pltpu.VMEM((NCH, CHUNK), jnp.int32),
            pltpu.VMEM((CHUNK, H), jnp.float32),
            pltpu.VMEM_SHARED((NPAD, H), jnp.float32),
            pltpu.VMEM_SHARED((NPAD, H), jnp.float32),
        ],
        name="gcn_msg_pass",
    )(_mp_body)
    return f(y, srcp, dstp, zeros2d)


def _sel_body(t_hbm, idx_hbm, out_hbm, idxv, rows, sem):
    c = lax.axis_index("c")
    s = lax.axis_index("s")
    w = c * NS + s
    base = w * BPW
    pltpu.sync_copy(idx_hbm.at[pl.ds(base, BPW)], idxv)
    pltpu.async_copy(t_hbm.at[idxv], rows, sem).wait()
    pltpu.sync_copy(rows, out_hbm.at[pl.ds(base, BPW)])


def _sel_call(t, idx):
    f = functools.partial(
        pl.kernel,
        out_type=jax.ShapeDtypeStruct((1024, H), jnp.float32),
        mesh=_sc_mesh(),
        scratch_types=[
            pltpu.VMEM((BPW,), jnp.int32),
            pltpu.VMEM((BPW, H), jnp.float32),
            pltpu.SemaphoreType.DMA,
        ],
        name="gcn_select",
    )(_sel_body)
    return f(t, idx)


def _dinv_of(d0_ref, d1_ref):
    deg = d0_ref[0, 0, :] + d1_ref[0, 0, :] + 1.0
    return lax.rsqrt(deg)


def _ab_body(x_ref, w0_ref, d0_ref, d1_ref, y0_ref):
    dinv = _dinv_of(d0_ref, d1_ref)
    xw = jnp.dot(x_ref[...], w0_ref[...], preferred_element_type=jnp.float32)
    y0_ref[...] = xw * dinv[:, None]


def _ab_call(xp, W0, d0, d1):
    return pl.pallas_call(
        _ab_body,
        grid=(NBLK,),
        in_specs=[
            pl.BlockSpec((BR, F0), lambda i: (i, 0)),
            pl.BlockSpec((F0, H), lambda i: (0, 0)),
            pl.BlockSpec((1, 1, BR), lambda i: (i, 0, 0)),
            pl.BlockSpec((1, 1, BR), lambda i: (i, 0, 0)),
        ],
        out_specs=pl.BlockSpec((BR, H), lambda i: (i, 0)),
        out_shape=jax.ShapeDtypeStruct((NPAD, H), jnp.float32),
        name="gcn_xw_dinv",
    )(xp, W0, d0, d1)


def _post_body(s0_ref, s1_ref, y_ref, d0_ref, d1_ref, b_ref, t_ref, st_ref):
    i = pl.program_id(0)
    dinv = _dinv_of(d0_ref, d1_ref)
    tot = (s0_ref[...] + s1_ref[...] + y_ref[...]) * dinv[:, None] + b_ref[...]
    t = jnp.where(tot > 0, tot, SLOPE * tot)
    t_ref[...] = t
    rid = i * BR + lax.broadcasted_iota(jnp.int32, (BR, 1), 0)
    tm = jnp.where(rid < N, t, 0.0)
    s1 = jnp.sum(tm, axis=0, keepdims=True)
    s2 = jnp.sum(tm * tm, axis=0, keepdims=True)
    st = jnp.concatenate([s1, s2, jnp.zeros((6, H), jnp.float32)], axis=0)

    @pl.when(i == 0)
    def _():
        st_ref[...] = st

    @pl.when(i != 0)
    def _():
        st_ref[...] += st


def _post_call(scat0, scat1, y, d0, d1, b):
    return pl.pallas_call(
        _post_body,
        grid=(NBLK,),
        in_specs=[
            pl.BlockSpec((BR, H), lambda i: (i, 0)),
            pl.BlockSpec((BR, H), lambda i: (i, 0)),
            pl.BlockSpec((BR, H), lambda i: (i, 0)),
            pl.BlockSpec((1, 1, BR), lambda i: (i, 0, 0)),
            pl.BlockSpec((1, 1, BR), lambda i: (i, 0, 0)),
            pl.BlockSpec((1, H), lambda i: (0, 0)),
        ],
        out_specs=[
            pl.BlockSpec((BR, H), lambda i: (i, 0)),
            pl.BlockSpec((8, H), lambda i: (0, 0)),
        ],
        out_shape=[
            jax.ShapeDtypeStruct((NPAD, H), jnp.float32),
            jax.ShapeDtypeStruct((8, H), jnp.float32),
        ],
        name="gcn_post_stats",
    )(scat0, scat1, y, d0, d1, b)


def _bn_mm_body(t_ref, st_ref, g_ref, be_ref, w1_ref, d0_ref, d1_ref, y1_ref):
    i = pl.program_id(0)
    mean = st_ref[0:1, :] / float(N)
    var = st_ref[1:2, :] / float(N) - mean * mean
    alpha = g_ref[...] * lax.rsqrt(var + EPS)
    h = (t_ref[...] - mean) * alpha + be_ref[...]
    dinv = _dinv_of(d0_ref, d1_ref)
    y1 = jnp.dot(h, w1_ref[...], preferred_element_type=jnp.float32)
    y1 = y1 * dinv[:, None]
    rid = i * BR + lax.broadcasted_iota(jnp.int32, (BR, 1), 0)
    y1_ref[...] = jnp.where(rid < N, y1, 0.0)


def _bn_mm_call(t0, st0, g, be, W1, d0, d1):
    return pl.pallas_call(
        _bn_mm_body,
        grid=(NBLK,),
        in_specs=[
            pl.BlockSpec((BR, H), lambda i: (i, 0)),
            pl.BlockSpec((8, H), lambda i: (0, 0)),
            pl.BlockSpec((1, H), lambda i: (0, 0)),
            pl.BlockSpec((1, H), lambda i: (0, 0)),
            pl.BlockSpec((H, H), lambda i: (0, 0)),
            pl.BlockSpec((1, 1, BR), lambda i: (i, 0, 0)),
            pl.BlockSpec((1, 1, BR), lambda i: (i, 0, 0)),
        ],
        out_specs=pl.BlockSpec((BR, H), lambda i: (i, 0)),
        out_shape=jax.ShapeDtypeStruct((NPAD, H), jnp.float32),
        name="gcn_bn_mm",
    )(t0, st0, g, be, W1, d0, d1)


def _head_body(tsel_ref, st_ref, g_ref, be_ref, wm_ref, bm_ref, h_ref, o_ref):
    mean = st_ref[0:1, :] / float(N)
    var = st_ref[1:2, :] / float(N) - mean * mean
    alpha = g_ref[...] * lax.rsqrt(var + EPS)
    h = (tsel_ref[...] - mean) * alpha + be_ref[...]
    h_ref[...] = h
    z = jnp.dot(h, wm_ref[...], preferred_element_type=jnp.float32) + bm_ref[...]
    o_ref[...] = jax.nn.sigmoid(z)


def _head_call(tsel, st1, g, be, Wmp, bmp):
    return pl.pallas_call(
        _head_body,
        grid=(1,),
        in_specs=[
            pl.BlockSpec((1024, H), lambda i: (0, 0)),
            pl.BlockSpec((8, H), lambda i: (0, 0)),
            pl.BlockSpec((1, H), lambda i: (0, 0)),
            pl.BlockSpec((1, H), lambda i: (0, 0)),
            pl.BlockSpec((H, 128), lambda i: (0, 0)),
            pl.BlockSpec((1, 128), lambda i: (0, 0)),
        ],
        out_specs=[
            pl.BlockSpec((1024, H), lambda i: (0, 0)),
            pl.BlockSpec((1024, 128), lambda i: (0, 0)),
        ],
        out_shape=[
            jax.ShapeDtypeStruct((1024, H), jnp.float32),
            jax.ShapeDtypeStruct((1024, 128), jnp.float32),
        ],
        name="gcn_head",
    )(tsel, st1, g, be, Wmp, bmp)


def kernel(x, edge_index, idx, W0, b0, g0, be0, W1, b1, g1, be1, Wm, bm):
    xp = jnp.pad(x, ((0, NPAD - N), (0, 0)))
    pad = jnp.full((EPAD - E,), -1, jnp.int32)
    srcp = jnp.concatenate([edge_index[0], pad]).reshape(NW, NCH, CHUNK)
    dstp = jnp.concatenate([edge_index[1], pad]).reshape(NW, NCH, CHUNK)
    zeros2d = jnp.zeros((NPAD, H), jnp.float32)

    degp = _deg_call(dstp)
    d0 = degp[0].reshape(NBLK, 1, BR)
    d1 = degp[1].reshape(NBLK, 1, BR)

    y0 = _ab_call(xp, W0, d0, d1)
    scat0 = _mp_call(y0, srcp, dstp, zeros2d)
    t0, st0 = _post_call(scat0[0], scat0[1], y0, d0, d1, b0.reshape(1, H))
    y1 = _bn_mm_call(t0, st0, g0.reshape(1, H), be0.reshape(1, H), W1, d0, d1)
    scat1 = _mp_call(y1, srcp, dstp, zeros2d)
    t1, st1 = _post_call(scat1[0], scat1[1], y1, d0, d1, b1.reshape(1, H))
    tsel = _sel_call(t1, idx)

    Wmp = jnp.pad(Wm, ((0, 0), (0, 128 - Wm.shape[1])))
    bmp = jnp.pad(bm, (0, 128 - bm.shape[0])).reshape(1, 128)
    h, o = _head_call(tsel, st1, g1.reshape(1, H), be1.reshape(1, H), Wmp, bmp)
    return (h, o[:, : Wm.shape[1]])


# speedup vs baseline: 22.6911x; 22.6911x over previous
import functools

import jax
import jax.numpy as jnp
from jax import lax
from jax.experimental import pallas as pl
from jax.experimental.pallas import tpu as pltpu
from jax.experimental.pallas import tpu_sc as plsc

N = 10000
NPAD = 10240
F0 = 128
H = 64
E = 320000
EPS = 1e-5
SLOPE = 0.01

NC = 2
NS = 16
NW = NC * NS
CHUNK = 128
NCH = -(-E // (NW * CHUNK))
EPW = NCH * CHUNK
EPAD = EPW * NW
ZPT = NPAD // NS
BR = 256
NBLK = NPAD // BR
BPW = 1024 // NW


def _sc_mesh():
    return plsc.VectorSubcoreMesh(
        core_axis_name="c", subcore_axis_name="s", num_cores=NC, num_subcores=NS
    )


_SC_PARAMS = pltpu.CompilerParams(use_tc_tiling_on_sc=False)


def _deg_body(dst_hbm, out_hbm, dstv, onesv, zv, deg_sp):
    c = lax.axis_index("c")
    s = lax.axis_index("s")
    w = c * NS + s
    for k in range(CHUNK // 16):
        onesv[pl.ds(k * 16, 16)] = jnp.ones((16,), jnp.float32)

    def zbody(k, _):
        zv[pl.ds(k * 16, 16)] = jnp.zeros((16,), jnp.float32)
        return 0

    lax.fori_loop(0, ZPT // 16, zbody, 0)
    pltpu.sync_copy(zv, deg_sp.at[pl.ds(s * ZPT, ZPT)])
    pltpu.sync_copy(dst_hbm.at[w], dstv)
    plsc.subcore_barrier()

    def body(j, _):
        pltpu.sync_copy(
            onesv,
            deg_sp.at[plsc.Indices(dstv.at[j], ignored_value=-1)],
            add=True,
        )
        return 0

    lax.fori_loop(0, NCH, body, 0)
    plsc.subcore_barrier()
    pltpu.sync_copy(deg_sp.at[pl.ds(s * ZPT, ZPT)], zv)
    pltpu.sync_copy(zv, out_hbm.at[c, pl.ds(s * ZPT, ZPT)])


def _deg_call(dstp):
    f = functools.partial(
        pl.kernel,
        out_type=jax.ShapeDtypeStruct((NC, NPAD), jnp.float32),
        mesh=_sc_mesh(),
        scratch_types=[
            pltpu.VMEM((NCH, CHUNK), jnp.int32),
            pltpu.VMEM((CHUNK,), jnp.float32),
            pltpu.VMEM((ZPT,), jnp.float32),
            pltpu.VMEM_SHARED((NPAD,), jnp.float32),
        ],
        compiler_params=_SC_PARAMS,
        name="gcn_deg",
    )(_deg_body)
    return f(dstp)


def _mp_body(y_hbm, src_hbm, dst_hbm, zero_hbm, out_hbm,
             srcv, dstv, rows, acc_sp, sem):
    c = lax.axis_index("c")
    s = lax.axis_index("s")
    w = c * NS + s
    r0 = s * ZPT
    pltpu.sync_copy(zero_hbm.at[pl.ds(r0, ZPT)], acc_sp.at[pl.ds(r0, ZPT)])
    pltpu.sync_copy(src_hbm.at[w], srcv)
    pltpu.sync_copy(dst_hbm.at[w], dstv)
    plsc.subcore_barrier()

    def body(j, _):
        pltpu.async_copy(
            y_hbm.at[plsc.Indices(srcv.at[j], ignored_value=-1)], rows, sem
        ).wait()
        pltpu.sync_copy(
            rows,
            acc_sp.at[plsc.Indices(dstv.at[j], ignored_value=-1)],
            add=True,
        )
        return 0

    lax.fori_loop(0, NCH, body, 0)
    plsc.subcore_barrier()
    pltpu.sync_copy(acc_sp.at[pl.ds(r0, ZPT)], out_hbm.at[c, pl.ds(r0, ZPT)])


def _mp_call(y, srcp, dstp, zeros2d):
    f = functools.partial(
        pl.kernel,
        out_type=jax.ShapeDtypeStruct((NC, NPAD, H), jnp.float32),
        mesh=_sc_mesh(),
        scratch_types=[
            pltpu.VMEM((NCH, CHUNK), jnp.int32),
            pltpu.VMEM((NCH, CHUNK), jnp.int32),
            pltpu.VMEM((CHUNK, H), jnp.float32),
            pltpu.VMEM_SHARED((NPAD, H), jnp.float32),
            pltpu.SemaphoreType.DMA,
        ],
        compiler_params=_SC_PARAMS,
        name="gcn_msg_pass",
    )(_mp_body)
    return f(y, srcp, dstp, zeros2d)


def _sel_body(t_hbm, idx_hbm, out_hbm, idxv, rows, sem):
    c = lax.axis_index("c")
    s = lax.axis_index("s")
    w = c * NS + s
    base = w * BPW
    pltpu.sync_copy(idx_hbm.at[pl.ds(base, BPW)], idxv)
    pltpu.async_copy(t_hbm.at[idxv], rows, sem).wait()
    pltpu.sync_copy(rows, out_hbm.at[pl.ds(base, BPW)])


def _sel_call(t, idx):
    f = functools.partial(
        pl.kernel,
        out_type=jax.ShapeDtypeStruct((1024, H), jnp.float32),
        mesh=_sc_mesh(),
        scratch_types=[
            pltpu.VMEM((BPW,), jnp.int32),
            pltpu.VMEM((BPW, H), jnp.float32),
            pltpu.SemaphoreType.DMA,
        ],
        compiler_params=_SC_PARAMS,
        name="gcn_select",
    )(_sel_body)
    return f(t, idx)


def _dinv_of(d0_ref, d1_ref):
    deg = d0_ref[0, 0, :] + d1_ref[0, 0, :] + 1.0
    return lax.rsqrt(deg)


def _ab_body(x_ref, w0_ref, d0_ref, d1_ref, y0_ref):
    dinv = _dinv_of(d0_ref, d1_ref)
    xw = jnp.dot(x_ref[...], w0_ref[...], preferred_element_type=jnp.float32)
    y0_ref[...] = xw * dinv[:, None]


def _ab_call(xp, W0, d0, d1):
    return pl.pallas_call(
        _ab_body,
        grid=(NBLK,),
        in_specs=[
            pl.BlockSpec((BR, F0), lambda i: (i, 0)),
            pl.BlockSpec((F0, H), lambda i: (0, 0)),
            pl.BlockSpec((1, 1, BR), lambda i: (i, 0, 0)),
            pl.BlockSpec((1, 1, BR), lambda i: (i, 0, 0)),
        ],
        out_specs=pl.BlockSpec((BR, H), lambda i: (i, 0)),
        out_shape=jax.ShapeDtypeStruct((NPAD, H), jnp.float32),
        name="gcn_xw_dinv",
    )(xp, W0, d0, d1)


def _post_body(s0_ref, s1_ref, y_ref, d0_ref, d1_ref, b_ref, t_ref, st_ref):
    i = pl.program_id(0)
    dinv = _dinv_of(d0_ref, d1_ref)
    tot = (s0_ref[...] + s1_ref[...] + y_ref[...]) * dinv[:, None] + b_ref[...]
    t = jnp.where(tot > 0, tot, SLOPE * tot)
    t_ref[...] = t
    rid = i * BR + lax.broadcasted_iota(jnp.int32, (BR, 1), 0)
    tm = jnp.where(rid < N, t, 0.0)
    s1 = jnp.sum(tm, axis=0, keepdims=True)
    s2 = jnp.sum(tm * tm, axis=0, keepdims=True)
    st = jnp.concatenate([s1, s2, jnp.zeros((6, H), jnp.float32)], axis=0)

    @pl.when(i == 0)
    def _():
        st_ref[...] = st

    @pl.when(i != 0)
    def _():
        st_ref[...] += st


def _post_call(scat0, scat1, y, d0, d1, b):
    return pl.pallas_call(
        _post_body,
        grid=(NBLK,),
        in_specs=[
            pl.BlockSpec((BR, H), lambda i: (i, 0)),
            pl.BlockSpec((BR, H), lambda i: (i, 0)),
            pl.BlockSpec((BR, H), lambda i: (i, 0)),
            pl.BlockSpec((1, 1, BR), lambda i: (i, 0, 0)),
            pl.BlockSpec((1, 1, BR), lambda i: (i, 0, 0)),
            pl.BlockSpec((1, H), lambda i: (0, 0)),
        ],
        out_specs=[
            pl.BlockSpec((BR, H), lambda i: (i, 0)),
            pl.BlockSpec((8, H), lambda i: (0, 0)),
        ],
        out_shape=[
            jax.ShapeDtypeStruct((NPAD, H), jnp.float32),
            jax.ShapeDtypeStruct((8, H), jnp.float32),
        ],
        name="gcn_post_stats",
    )(scat0, scat1, y, d0, d1, b)


def _bn_mm_body(t_ref, st_ref, g_ref, be_ref, w1_ref, d0_ref, d1_ref, y1_ref):
    i = pl.program_id(0)
    mean = st_ref[0:1, :] / float(N)
    var = st_ref[1:2, :] / float(N) - mean * mean
    alpha = g_ref[...] * lax.rsqrt(var + EPS)
    h = (t_ref[...] - mean) * alpha + be_ref[...]
    dinv = _dinv_of(d0_ref, d1_ref)
    y1 = jnp.dot(h, w1_ref[...], preferred_element_type=jnp.float32)
    y1 = y1 * dinv[:, None]
    rid = i * BR + lax.broadcasted_iota(jnp.int32, (BR, 1), 0)
    y1_ref[...] = jnp.where(rid < N, y1, 0.0)


def _bn_mm_call(t0, st0, g, be, W1, d0, d1):
    return pl.pallas_call(
        _bn_mm_body,
        grid=(NBLK,),
        in_specs=[
            pl.BlockSpec((BR, H), lambda i: (i, 0)),
            pl.BlockSpec((8, H), lambda i: (0, 0)),
            pl.BlockSpec((1, H), lambda i: (0, 0)),
            pl.BlockSpec((1, H), lambda i: (0, 0)),
            pl.BlockSpec((H, H), lambda i: (0, 0)),
            pl.BlockSpec((1, 1, BR), lambda i: (i, 0, 0)),
            pl.BlockSpec((1, 1, BR), lambda i: (i, 0, 0)),
        ],
        out_specs=pl.BlockSpec((BR, H), lambda i: (i, 0)),
        out_shape=jax.ShapeDtypeStruct((NPAD, H), jnp.float32),
        name="gcn_bn_mm",
    )(t0, st0, g, be, W1, d0, d1)


def _head_body(tsel_ref, st_ref, g_ref, be_ref, wm_ref, bm_ref, h_ref, o_ref):
    mean = st_ref[0:1, :] / float(N)
    var = st_ref[1:2, :] / float(N) - mean * mean
    alpha = g_ref[...] * lax.rsqrt(var + EPS)
    h = (tsel_ref[...] - mean) * alpha + be_ref[...]
    h_ref[...] = h
    z = jnp.dot(h, wm_ref[...], preferred_element_type=jnp.float32) + bm_ref[...]
    o_ref[...] = jax.nn.sigmoid(z)


def _head_call(tsel, st1, g, be, Wmp, bmp):
    return pl.pallas_call(
        _head_body,
        grid=(1,),
        in_specs=[
            pl.BlockSpec((1024, H), lambda i: (0, 0)),
            pl.BlockSpec((8, H), lambda i: (0, 0)),
            pl.BlockSpec((1, H), lambda i: (0, 0)),
            pl.BlockSpec((1, H), lambda i: (0, 0)),
            pl.BlockSpec((H, 128), lambda i: (0, 0)),
            pl.BlockSpec((1, 128), lambda i: (0, 0)),
        ],
        out_specs=[
            pl.BlockSpec((1024, H), lambda i: (0, 0)),
            pl.BlockSpec((1024, 128), lambda i: (0, 0)),
        ],
        out_shape=[
            jax.ShapeDtypeStruct((1024, H), jnp.float32),
            jax.ShapeDtypeStruct((1024, 128), jnp.float32),
        ],
        name="gcn_head",
    )(tsel, st1, g, be, Wmp, bmp)


def kernel(x, edge_index, idx, W0, b0, g0, be0, W1, b1, g1, be1, Wm, bm):
    xp = jnp.pad(x, ((0, NPAD - N), (0, 0)))
    pad = jnp.full((EPAD - E,), -1, jnp.int32)
    srcp = jnp.concatenate([edge_index[0], pad]).reshape(NW, NCH, CHUNK)
    dstp = jnp.concatenate([edge_index[1], pad]).reshape(NW, NCH, CHUNK)
    zeros2d = jnp.zeros((NPAD, H), jnp.float32)

    degp = _deg_call(dstp)
    d0 = degp[0].reshape(NBLK, 1, BR)
    d1 = degp[1].reshape(NBLK, 1, BR)

    y0 = _ab_call(xp, W0, d0, d1)
    scat0 = _mp_call(y0, srcp, dstp, zeros2d)
    t0, st0 = _post_call(scat0[0], scat0[1], y0, d0, d1, b0.reshape(1, H))
    y1 = _bn_mm_call(t0, st0, g0.reshape(1, H), be0.reshape(1, H), W1, d0, d1)
    scat1 = _mp_call(y1, srcp, dstp, zeros2d)
    t1, st1 = _post_call(scat1[0], scat1[1], y1, d0, d1, b1.reshape(1, H))
    tsel = _sel_call(t1, idx)

    Wmp = jnp.pad(Wm, ((0, 0), (0, 128 - Wm.shape[1])))
    bmp = jnp.pad(bm, (0, 128 - bm.shape[0])).reshape(1, 128)
    h, o = _head_call(tsel, st1, g1.reshape(1, H), be1.reshape(1, H), Wmp, bmp)
    return (h, o[:, : Wm.shape[1]])


# speedup vs baseline: 25.7210x; 1.1335x over previous
import functools

import jax
import jax.numpy as jnp
from jax import lax
from jax.experimental import pallas as pl
from jax.experimental.pallas import tpu as pltpu
from jax.experimental.pallas import tpu_sc as plsc

N = 10000
NPAD = 10240
F0 = 128
H = 64
E = 320000
EPS = 1e-5
SLOPE = 0.01

NC = 2
NS = 16
NW = NC * NS
CHUNK = 128
NCH = -(-E // (NW * CHUNK))
NCH += NCH % 2
EPW = NCH * CHUNK
EPAD = EPW * NW
ZPT = NPAD // NS
BR = 256
NBLK = NPAD // BR
BPW = 1024 // NW


def _sc_mesh():
    return plsc.VectorSubcoreMesh(
        core_axis_name="c", subcore_axis_name="s", num_cores=NC, num_subcores=NS
    )


_SC_PARAMS = pltpu.CompilerParams(use_tc_tiling_on_sc=False)


def _deg_body(dst_hbm, out_hbm, dstv, onesv, zv, deg_sp):
    c = lax.axis_index("c")
    s = lax.axis_index("s")
    w = c * NS + s
    for k in range(CHUNK // 16):
        onesv[pl.ds(k * 16, 16)] = jnp.ones((16,), jnp.float32)

    def zbody(k, _):
        zv[pl.ds(k * 16, 16)] = jnp.zeros((16,), jnp.float32)
        return 0

    lax.fori_loop(0, ZPT // 16, zbody, 0)
    pltpu.sync_copy(zv, deg_sp.at[pl.ds(s * ZPT, ZPT)])
    pltpu.sync_copy(dst_hbm.at[w], dstv)
    plsc.subcore_barrier()

    def body(j, _):
        pltpu.sync_copy(
            onesv,
            deg_sp.at[plsc.Indices(dstv.at[j], ignored_value=-1)],
            add=True,
        )
        return 0

    lax.fori_loop(0, NCH, body, 0)
    plsc.subcore_barrier()
    pltpu.sync_copy(deg_sp.at[pl.ds(s * ZPT, ZPT)], zv)
    pltpu.sync_copy(zv, out_hbm.at[c, pl.ds(s * ZPT, ZPT)])


def _deg_call(dstp):
    f = functools.partial(
        pl.kernel,
        out_type=jax.ShapeDtypeStruct((NC, NPAD), jnp.float32),
        mesh=_sc_mesh(),
        scratch_types=[
            pltpu.VMEM((NCH, CHUNK), jnp.int32),
            pltpu.VMEM((CHUNK,), jnp.float32),
            pltpu.VMEM((ZPT,), jnp.float32),
            pltpu.VMEM_SHARED((NPAD,), jnp.float32),
        ],
        compiler_params=_SC_PARAMS,
        name="gcn_deg",
    )(_deg_body)
    return f(dstp)


def _mp_body(y_hbm, src_hbm, dst_hbm, zero_hbm, out_hbm,
             srcv, dstv, rows_a, rows_b, acc_sp, sem_ga, sem_gb):
    c = lax.axis_index("c")
    s = lax.axis_index("s")
    w = c * NS + s
    r0 = s * ZPT
    pltpu.sync_copy(zero_hbm.at[pl.ds(r0, ZPT)], acc_sp.at[pl.ds(r0, ZPT)])
    pltpu.sync_copy(src_hbm.at[w], srcv)
    pltpu.sync_copy(dst_hbm.at[w], dstv)
    plsc.subcore_barrier()

    def gather(j, rows, sem):
        return pltpu.async_copy(
            y_hbm.at[plsc.Indices(srcv.at[j], ignored_value=-1)], rows, sem
        )

    def scatter(j, rows, sem):
        return pltpu.async_copy(
            rows,
            acc_sp.at[plsc.Indices(dstv.at[j], ignored_value=-1)],
            sem,
            add=True,
        )

    def gwait(j, rows, sem):
        pltpu.make_async_copy(
            y_hbm.at[plsc.Indices(srcv.at[j], ignored_value=-1)], rows, sem
        ).wait()

    def scatter_sync(j, rows):
        pltpu.sync_copy(
            rows,
            acc_sp.at[plsc.Indices(dstv.at[j], ignored_value=-1)],
            add=True,
        )

    gather(0, rows_a, sem_ga)

    def body(k, _):
        j0 = 2 * k
        gwait(j0, rows_a, sem_ga)
        gather(j0 + 1, rows_b, sem_gb)
        scatter_sync(j0, rows_a)
        gwait(j0 + 1, rows_b, sem_gb)
        gather(j0 + 2, rows_a, sem_ga)
        scatter_sync(j0 + 1, rows_b)
        return 0

    lax.fori_loop(0, NCH // 2 - 1, body, 0)
    jl = NCH - 2
    gwait(jl, rows_a, sem_ga)
    gather(jl + 1, rows_b, sem_gb)
    scatter_sync(jl, rows_a)
    gwait(jl + 1, rows_b, sem_gb)
    scatter_sync(jl + 1, rows_b)

    plsc.subcore_barrier()
    pltpu.sync_copy(acc_sp.at[pl.ds(r0, ZPT)], out_hbm.at[c, pl.ds(r0, ZPT)])


def _mp_call(y, srcp, dstp, zeros2d):
    f = functools.partial(
        pl.kernel,
        out_type=jax.ShapeDtypeStruct((NC, NPAD, H), jnp.float32),
        mesh=_sc_mesh(),
        scratch_types=[
            pltpu.VMEM((NCH, CHUNK), jnp.int32),
            pltpu.VMEM((NCH, CHUNK), jnp.int32),
            pltpu.VMEM((CHUNK, H), jnp.float32),
            pltpu.VMEM((CHUNK, H), jnp.float32),
            pltpu.VMEM_SHARED((NPAD, H), jnp.float32),
            pltpu.SemaphoreType.DMA,
            pltpu.SemaphoreType.DMA,
        ],
        compiler_params=_SC_PARAMS,
        name="gcn_msg_pass",
    )(_mp_body)
    return f(y, srcp, dstp, zeros2d)


def _sel_body(t_hbm, idx_hbm, out_hbm, idxv, rows, sem):
    c = lax.axis_index("c")
    s = lax.axis_index("s")
    w = c * NS + s
    base = w * BPW
    pltpu.sync_copy(idx_hbm.at[pl.ds(base, BPW)], idxv)
    pltpu.async_copy(t_hbm.at[idxv], rows, sem).wait()
    pltpu.sync_copy(rows, out_hbm.at[pl.ds(base, BPW)])


def _sel_call(t, idx):
    f = functools.partial(
        pl.kernel,
        out_type=jax.ShapeDtypeStruct((1024, H), jnp.float32),
        mesh=_sc_mesh(),
        scratch_types=[
            pltpu.VMEM((BPW,), jnp.int32),
            pltpu.VMEM((BPW, H), jnp.float32),
            pltpu.SemaphoreType.DMA,
        ],
        compiler_params=_SC_PARAMS,
        name="gcn_select",
    )(_sel_body)
    return f(t, idx)


def _dinv_of(d0_ref, d1_ref):
    deg = d0_ref[0, 0, :] + d1_ref[0, 0, :] + 1.0
    return lax.rsqrt(deg)


def _ab_body(x_ref, w0_ref, d0_ref, d1_ref, y0_ref):
    dinv = _dinv_of(d0_ref, d1_ref)
    xw = jnp.dot(x_ref[...], w0_ref[...], preferred_element_type=jnp.float32)
    y0_ref[...] = xw * dinv[:, None]


def _ab_call(xp, W0, d0, d1):
    return pl.pallas_call(
        _ab_body,
        grid=(NBLK,),
        in_specs=[
            pl.BlockSpec((BR, F0), lambda i: (i, 0)),
            pl.BlockSpec((F0, H), lambda i: (0, 0)),
            pl.BlockSpec((1, 1, BR), lambda i: (i, 0, 0)),
            pl.BlockSpec((1, 1, BR), lambda i: (i, 0, 0)),
        ],
        out_specs=pl.BlockSpec((BR, H), lambda i: (i, 0)),
        out_shape=jax.ShapeDtypeStruct((NPAD, H), jnp.float32),
        name="gcn_xw_dinv",
    )(xp, W0, d0, d1)


def _post_body(s0_ref, s1_ref, y_ref, d0_ref, d1_ref, b_ref, t_ref, st_ref):
    i = pl.program_id(0)
    dinv = _dinv_of(d0_ref, d1_ref)
    tot = (s0_ref[...] + s1_ref[...] + y_ref[...]) * dinv[:, None] + b_ref[...]
    t = jnp.where(tot > 0, tot, SLOPE * tot)
    t_ref[...] = t
    rid = i * BR + lax.broadcasted_iota(jnp.int32, (BR, 1), 0)
    tm = jnp.where(rid < N, t, 0.0)
    s1 = jnp.sum(tm, axis=0, keepdims=True)
    s2 = jnp.sum(tm * tm, axis=0, keepdims=True)
    st = jnp.concatenate([s1, s2, jnp.zeros((6, H), jnp.float32)], axis=0)

    @pl.when(i == 0)
    def _():
        st_ref[...] = st

    @pl.when(i != 0)
    def _():
        st_ref[...] += st


def _post_call(scat0, scat1, y, d0, d1, b):
    return pl.pallas_call(
        _post_body,
        grid=(NBLK,),
        in_specs=[
            pl.BlockSpec((BR, H), lambda i: (i, 0)),
            pl.BlockSpec((BR, H), lambda i: (i, 0)),
            pl.BlockSpec((BR, H), lambda i: (i, 0)),
            pl.BlockSpec((1, 1, BR), lambda i: (i, 0, 0)),
            pl.BlockSpec((1, 1, BR), lambda i: (i, 0, 0)),
            pl.BlockSpec((1, H), lambda i: (0, 0)),
        ],
        out_specs=[
            pl.BlockSpec((BR, H), lambda i: (i, 0)),
            pl.BlockSpec((8, H), lambda i: (0, 0)),
        ],
        out_shape=[
            jax.ShapeDtypeStruct((NPAD, H), jnp.float32),
            jax.ShapeDtypeStruct((8, H), jnp.float32),
        ],
        name="gcn_post_stats",
    )(scat0, scat1, y, d0, d1, b)


def _bn_mm_body(t_ref, st_ref, g_ref, be_ref, w1_ref, d0_ref, d1_ref, y1_ref):
    i = pl.program_id(0)
    mean = st_ref[0:1, :] / float(N)
    var = st_ref[1:2, :] / float(N) - mean * mean
    alpha = g_ref[...] * lax.rsqrt(var + EPS)
    h = (t_ref[...] - mean) * alpha + be_ref[...]
    dinv = _dinv_of(d0_ref, d1_ref)
    y1 = jnp.dot(h, w1_ref[...], preferred_element_type=jnp.float32)
    y1 = y1 * dinv[:, None]
    rid = i * BR + lax.broadcasted_iota(jnp.int32, (BR, 1), 0)
    y1_ref[...] = jnp.where(rid < N, y1, 0.0)


def _bn_mm_call(t0, st0, g, be, W1, d0, d1):
    return pl.pallas_call(
        _bn_mm_body,
        grid=(NBLK,),
        in_specs=[
            pl.BlockSpec((BR, H), lambda i: (i, 0)),
            pl.BlockSpec((8, H), lambda i: (0, 0)),
            pl.BlockSpec((1, H), lambda i: (0, 0)),
            pl.BlockSpec((1, H), lambda i: (0, 0)),
            pl.BlockSpec((H, H), lambda i: (0, 0)),
            pl.BlockSpec((1, 1, BR), lambda i: (i, 0, 0)),
            pl.BlockSpec((1, 1, BR), lambda i: (i, 0, 0)),
        ],
        out_specs=pl.BlockSpec((BR, H), lambda i: (i, 0)),
        out_shape=jax.ShapeDtypeStruct((NPAD, H), jnp.float32),
        name="gcn_bn_mm",
    )(t0, st0, g, be, W1, d0, d1)


def _head_body(tsel_ref, st_ref, g_ref, be_ref, wm_ref, bm_ref, h_ref, o_ref):
    mean = st_ref[0:1, :] / float(N)
    var = st_ref[1:2, :] / float(N) - mean * mean
    alpha = g_ref[...] * lax.rsqrt(var + EPS)
    h = (tsel_ref[...] - mean) * alpha + be_ref[...]
    h_ref[...] = h
    z = jnp.dot(h, wm_ref[...], preferred_element_type=jnp.float32) + bm_ref[...]
    o_ref[...] = jax.nn.sigmoid(z)


def _head_call(tsel, st1, g, be, Wmp, bmp):
    return pl.pallas_call(
        _head_body,
        grid=(1,),
        in_specs=[
            pl.BlockSpec((1024, H), lambda i: (0, 0)),
            pl.BlockSpec((8, H), lambda i: (0, 0)),
            pl.BlockSpec((1, H), lambda i: (0, 0)),
            pl.BlockSpec((1, H), lambda i: (0, 0)),
            pl.BlockSpec((H, 128), lambda i: (0, 0)),
            pl.BlockSpec((1, 128), lambda i: (0, 0)),
        ],
        out_specs=[
            pl.BlockSpec((1024, H), lambda i: (0, 0)),
            pl.BlockSpec((1024, 128), lambda i: (0, 0)),
        ],
        out_shape=[
            jax.ShapeDtypeStruct((1024, H), jnp.float32),
            jax.ShapeDtypeStruct((1024, 128), jnp.float32),
        ],
        name="gcn_head",
    )(tsel, st1, g, be, Wmp, bmp)


def kernel(x, edge_index, idx, W0, b0, g0, be0, W1, b1, g1, be1, Wm, bm):
    xp = jnp.pad(x, ((0, NPAD - N), (0, 0)))
    pad = jnp.full((EPAD - E,), -1, jnp.int32)
    srcp = jnp.concatenate([edge_index[0], pad]).reshape(NW, NCH, CHUNK)
    dstp = jnp.concatenate([edge_index[1], pad]).reshape(NW, NCH, CHUNK)
    zeros2d = jnp.zeros((NPAD, H), jnp.float32)

    degp = _deg_call(dstp)
    d0 = degp[0].reshape(NBLK, 1, BR)
    d1 = degp[1].reshape(NBLK, 1, BR)

    y0 = _ab_call(xp, W0, d0, d1)
    scat0 = _mp_call(y0, srcp, dstp, zeros2d)
    t0, st0 = _post_call(scat0[0], scat0[1], y0, d0, d1, b0.reshape(1, H))
    y1 = _bn_mm_call(t0, st0, g0.reshape(1, H), be0.reshape(1, H), W1, d0, d1)
    scat1 = _mp_call(y1, srcp, dstp, zeros2d)
    t1, st1 = _post_call(scat1[0], scat1[1], y1, d0, d1, b1.reshape(1, H))
    tsel = _sel_call(t1, idx)

    Wmp = jnp.pad(Wm, ((0, 0), (0, 128 - Wm.shape[1])))
    bmp = jnp.pad(bm, (0, 128 - bm.shape[0])).reshape(1, 128)
    h, o = _head_call(tsel, st1, g1.reshape(1, H), be1.reshape(1, H), Wmp, bmp)
    return (h, o[:, : Wm.shape[1]])


# speedup vs baseline: 33.4709x; 1.3013x over previous
import functools

import jax
import jax.numpy as jnp
from jax import lax
from jax.experimental import pallas as pl
from jax.experimental.pallas import tpu as pltpu
from jax.experimental.pallas import tpu_sc as plsc

N = 10000
NP2 = N // 2
F0 = 128
H = 64
E = 320000
EPS = 1e-5
SLOPE = 0.01

NC = 2
NS = 16
NW = NC * NS
CHUNK = 125
NCH = E // (NW * CHUNK)
RPT = N // NS
DEG_R = 640
BR = 1000
NBLK = NP2 // BR
BPW = 1024 // NW


def _sc_mesh():
    return plsc.VectorSubcoreMesh(
        core_axis_name="c", subcore_axis_name="s", num_cores=NC, num_subcores=NS
    )


_SC_PARAMS = pltpu.CompilerParams(use_tc_tiling_on_sc=False)


def _deg_body(dst_hbm, out_hbm, dstv, onesv, zv, deg_sp):
    c = lax.axis_index("c")
    s = lax.axis_index("s")
    w = c * NS + s
    for k in range(128 // 16):
        onesv[pl.ds(k * 16, 16)] = jnp.ones((16,), jnp.float32)

    def zbody(k, _):
        zv[pl.ds(k * 16, 16)] = jnp.zeros((16,), jnp.float32)
        return 0

    lax.fori_loop(0, DEG_R // 16, zbody, 0)
    pltpu.sync_copy(zv, deg_sp.at[pl.ds(s * DEG_R, DEG_R)])
    pltpu.sync_copy(dst_hbm.at[w], dstv)
    plsc.subcore_barrier()

    def body(j, _):
        pltpu.sync_copy(
            onesv.at[pl.ds(0, CHUNK)], deg_sp.at[dstv.at[j]], add=True
        )
        return 0

    lax.fori_loop(0, NCH, body, 0)
    plsc.subcore_barrier()
    pltpu.sync_copy(deg_sp.at[pl.ds(s * DEG_R, DEG_R)], zv)
    pltpu.sync_copy(zv, out_hbm.at[c, s])


def _deg_call(dstp):
    f = functools.partial(
        pl.kernel,
        out_type=jax.ShapeDtypeStruct((NC, NS, DEG_R), jnp.float32),
        mesh=_sc_mesh(),
        scratch_types=[
            pltpu.VMEM((NCH, CHUNK), jnp.int32),
            pltpu.VMEM((128,), jnp.float32),
            pltpu.VMEM((DEG_R,), jnp.float32),
            pltpu.VMEM_SHARED((NS * DEG_R,), jnp.float32),
        ],
        compiler_params=_SC_PARAMS,
        name="gcn_deg",
    )(_deg_body)
    return f(dstp)


def _mp_body(y_hbm, src_hbm, dst_hbm, zero_hbm, out_hbm,
             srcv, dstv, rows_a, rows_b, acc_sp, sem_ga, sem_gb):
    c = lax.axis_index("c")
    s = lax.axis_index("s")
    w = c * NS + s
    r0 = s * RPT
    pltpu.sync_copy(zero_hbm.at[pl.ds(r0, RPT)], acc_sp.at[pl.ds(r0, RPT)])
    pltpu.sync_copy(src_hbm.at[w], srcv)
    pltpu.sync_copy(dst_hbm.at[w], dstv)
    plsc.subcore_barrier()

    def gather(j, rows, sem):
        return pltpu.async_copy(y_hbm.at[srcv.at[j]], rows, sem)

    def gwait(j, rows, sem):
        pltpu.make_async_copy(y_hbm.at[srcv.at[j]], rows, sem).wait()

    def scatter_sync(j, rows):
        pltpu.sync_copy(rows, acc_sp.at[dstv.at[j]], add=True)

    gather(0, rows_a, sem_ga)

    def body(k, _):
        j0 = 2 * k
        gwait(j0, rows_a, sem_ga)
        gather(j0 + 1, rows_b, sem_gb)
        scatter_sync(j0, rows_a)
        gwait(j0 + 1, rows_b, sem_gb)
        gather(j0 + 2, rows_a, sem_ga)
        scatter_sync(j0 + 1, rows_b)
        return 0

    lax.fori_loop(0, NCH // 2 - 1, body, 0)
    jl = NCH - 2
    gwait(jl, rows_a, sem_ga)
    gather(jl + 1, rows_b, sem_gb)
    scatter_sync(jl, rows_a)
    gwait(jl + 1, rows_b, sem_gb)
    scatter_sync(jl + 1, rows_b)

    plsc.subcore_barrier()
    pltpu.sync_copy(acc_sp.at[pl.ds(r0, RPT)], out_hbm.at[c, pl.ds(r0, RPT)])


def _mp_call(y, srcp, dstp, zeros2d):
    f = functools.partial(
        pl.kernel,
        out_type=jax.ShapeDtypeStruct((NC, N, H), jnp.float32),
        mesh=_sc_mesh(),
        scratch_types=[
            pltpu.VMEM((NCH, CHUNK), jnp.int32),
            pltpu.VMEM((NCH, CHUNK), jnp.int32),
            pltpu.VMEM((CHUNK, H), jnp.float32),
            pltpu.VMEM((CHUNK, H), jnp.float32),
            pltpu.VMEM_SHARED((N, H), jnp.float32),
            pltpu.SemaphoreType.DMA,
            pltpu.SemaphoreType.DMA,
        ],
        compiler_params=_SC_PARAMS,
        name="gcn_msg_pass",
    )(_mp_body)
    return f(y, srcp, dstp, zeros2d)


def _sel_body(t_hbm, idx_hbm, out_hbm, idxv, rows, sem):
    c = lax.axis_index("c")
    s = lax.axis_index("s")
    w = c * NS + s
    base = w * BPW
    pltpu.sync_copy(idx_hbm.at[pl.ds(base, BPW)], idxv)
    pltpu.async_copy(t_hbm.at[idxv], rows, sem).wait()
    pltpu.sync_copy(rows, out_hbm.at[pl.ds(base, BPW)])


def _sel_call(t, idx):
    f = functools.partial(
        pl.kernel,
        out_type=jax.ShapeDtypeStruct((1024, H), jnp.float32),
        mesh=_sc_mesh(),
        scratch_types=[
            pltpu.VMEM((BPW,), jnp.int32),
            pltpu.VMEM((BPW, H), jnp.float32),
            pltpu.SemaphoreType.DMA,
        ],
        compiler_params=_SC_PARAMS,
        name="gcn_select",
    )(_sel_body)
    return f(t, idx)


def _ds_of(dd_ref):
    deg = dd_ref[0] + dd_ref[1] + 1.0
    dinv = lax.rsqrt(deg)
    lane = lax.broadcasted_iota(jnp.int32, (1, F0), 1)
    return jnp.where(lane < H, dinv[:, 0:1], dinv[:, 1:2])


def _ab_body(x_ref, w2_ref, dd_ref, y0_ref):
    xw = jnp.dot(x_ref[...], w2_ref[...], preferred_element_type=jnp.float32)
    y0_ref[...] = xw * _ds_of(dd_ref)


def _ab_call(x2, W02, dd):
    return pl.pallas_call(
        _ab_body,
        grid=(NBLK,),
        in_specs=[
            pl.BlockSpec((BR, 2 * F0), lambda i: (i, 0)),
            pl.BlockSpec((2 * F0, F0), lambda i: (0, 0)),
            pl.BlockSpec((2, BR, 2), lambda i: (0, i, 0)),
        ],
        out_specs=pl.BlockSpec((BR, F0), lambda i: (i, 0)),
        out_shape=jax.ShapeDtypeStruct((NP2, F0), jnp.float32),
        name="gcn_xw_dinv",
    )(x2, W02, dd)


def _post_body(s0_ref, s1_ref, y_ref, dd_ref, b2_ref, t_ref, st_ref):
    i = pl.program_id(0)
    tot = (s0_ref[...] + s1_ref[...] + y_ref[...]) * _ds_of(dd_ref) + b2_ref[...]
    t = jnp.where(tot > 0, tot, SLOPE * tot)
    t_ref[...] = t
    s1 = jnp.sum(t, axis=0, keepdims=True)
    s2 = jnp.sum(t * t, axis=0, keepdims=True)
    st = jnp.concatenate([s1, s2, jnp.zeros((6, F0), jnp.float32)], axis=0)

    @pl.when(i == 0)
    def _():
        st_ref[...] = st

    @pl.when(i != 0)
    def _():
        st_ref[...] += st


def _post_call(scat0, scat1, y, dd, b2):
    return pl.pallas_call(
        _post_body,
        grid=(NBLK,),
        in_specs=[
            pl.BlockSpec((BR, F0), lambda i: (i, 0)),
            pl.BlockSpec((BR, F0), lambda i: (i, 0)),
            pl.BlockSpec((BR, F0), lambda i: (i, 0)),
            pl.BlockSpec((2, BR, 2), lambda i: (0, i, 0)),
            pl.BlockSpec((1, F0), lambda i: (0, 0)),
        ],
        out_specs=[
            pl.BlockSpec((BR, F0), lambda i: (i, 0)),
            pl.BlockSpec((8, F0), lambda i: (0, 0)),
        ],
        out_shape=[
            jax.ShapeDtypeStruct((NP2, F0), jnp.float32),
            jax.ShapeDtypeStruct((8, F0), jnp.float32),
        ],
        name="gcn_post_stats",
    )(scat0, scat1, y, dd, b2)


def _bn_affine(st_ref, g2_ref, be2_ref):
    s1 = st_ref[0:1, :]
    s2 = st_ref[1:2, :]
    m64 = (s1[:, :H] + s1[:, H:]) / float(N)
    q64 = (s2[:, :H] + s2[:, H:]) / float(N)
    var64 = q64 - m64 * m64
    mean2 = jnp.concatenate([m64, m64], axis=1)
    var2 = jnp.concatenate([var64, var64], axis=1)
    alpha2 = g2_ref[...] * lax.rsqrt(var2 + EPS)
    return mean2, alpha2, be2_ref[...]


def _bn_mm_body(t_ref, st_ref, g2_ref, be2_ref, w12_ref, dd_ref, y1_ref):
    mean2, alpha2, be2 = _bn_affine(st_ref, g2_ref, be2_ref)
    h = (t_ref[...] - mean2) * alpha2 + be2
    y1 = jnp.dot(h, w12_ref[...], preferred_element_type=jnp.float32)
    y1_ref[...] = y1 * _ds_of(dd_ref)


def _bn_mm_call(t0, st0, g2, be2, W12, dd):
    return pl.pallas_call(
        _bn_mm_body,
        grid=(NBLK,),
        in_specs=[
            pl.BlockSpec((BR, F0), lambda i: (i, 0)),
            pl.BlockSpec((8, F0), lambda i: (0, 0)),
            pl.BlockSpec((1, F0), lambda i: (0, 0)),
            pl.BlockSpec((1, F0), lambda i: (0, 0)),
            pl.BlockSpec((F0, F0), lambda i: (0, 0)),
            pl.BlockSpec((2, BR, 2), lambda i: (0, i, 0)),
        ],
        out_specs=pl.BlockSpec((BR, F0), lambda i: (i, 0)),
        out_shape=jax.ShapeDtypeStruct((NP2, F0), jnp.float32),
        name="gcn_bn_mm",
    )(t0, st0, g2, be2, W12, dd)


def _head_body(tsel_ref, st_ref, g2_ref, be2_ref, wm2_ref, bm2_ref,
               h_ref, o_ref):
    mean2, alpha2, be2 = _bn_affine(st_ref, g2_ref, be2_ref)
    h = (tsel_ref[...] - mean2) * alpha2 + be2
    h_ref[...] = h
    z = jnp.dot(h, wm2_ref[...], preferred_element_type=jnp.float32)
    o_ref[...] = jax.nn.sigmoid(z + bm2_ref[...])


def _head_call(tsel2, st1, g2, be2, Wm2, bm2):
    return pl.pallas_call(
        _head_body,
        grid=(1,),
        in_specs=[
            pl.BlockSpec((512, F0), lambda i: (0, 0)),
            pl.BlockSpec((8, F0), lambda i: (0, 0)),
            pl.BlockSpec((1, F0), lambda i: (0, 0)),
            pl.BlockSpec((1, F0), lambda i: (0, 0)),
            pl.BlockSpec((F0, F0), lambda i: (0, 0)),
            pl.BlockSpec((1, F0), lambda i: (0, 0)),
        ],
        out_specs=[
            pl.BlockSpec((512, F0), lambda i: (0, 0)),
            pl.BlockSpec((512, F0), lambda i: (0, 0)),
        ],
        out_shape=[
            jax.ShapeDtypeStruct((512, F0), jnp.float32),
            jax.ShapeDtypeStruct((512, F0), jnp.float32),
        ],
        name="gcn_head",
    )(tsel2, st1, g2, be2, Wm2, bm2)


def _blockdiag(W):
    k, m = W.shape
    z = jnp.zeros((k, m), W.dtype)
    return jnp.concatenate(
        [jnp.concatenate([W, z], axis=1), jnp.concatenate([z, W], axis=1)],
        axis=0,
    )


def _dup(v):
    return jnp.concatenate([v, v]).reshape(1, 2 * v.shape[0])


def kernel(x, edge_index, idx, W0, b0, g0, be0, W1, b1, g1, be1, Wm, bm):
    x2 = x.reshape(NP2, 2 * F0)
    srcp = edge_index[0].reshape(NW, NCH, CHUNK)
    dstp = edge_index[1].reshape(NW, NCH, CHUNK)
    zeros2d = jnp.zeros((N, H), jnp.float32)

    degp = _deg_call(dstp)
    dd = degp.reshape(NC, NS * DEG_R)[:, :N].reshape(NC, NP2, 2)

    W02 = _blockdiag(W0)
    y0 = _ab_call(x2, W02, dd)
    scat0 = _mp_call(y0.reshape(N, H), srcp, dstp, zeros2d)
    s2v = scat0.reshape(NC, NP2, F0)
    t0, st0 = _post_call(s2v[0], s2v[1], y0, dd, _dup(b0))
    y1 = _bn_mm_call(t0, st0, _dup(g0), _dup(be0), _blockdiag(W1), dd)
    scat1 = _mp_call(y1.reshape(N, H), srcp, dstp, zeros2d)
    s2v1 = scat1.reshape(NC, NP2, F0)
    t1, st1 = _post_call(s2v1[0], s2v1[1], y1, dd, _dup(b1))

    tsel = _sel_call(t1.reshape(N, H), idx)
    nm = Wm.shape[1]
    Wm2 = jnp.pad(_blockdiag(Wm), ((0, 0), (0, F0 - 2 * nm)))
    bm2 = jnp.pad(jnp.concatenate([bm, bm]), (0, F0 - 2 * nm)).reshape(1, F0)
    h2, o2 = _head_call(tsel.reshape(512, F0), st1, _dup(g1), _dup(be1),
                        Wm2, bm2)
    h = h2.reshape(1024, H)
    o = o2[:, : 2 * nm].reshape(1024, nm)
    return (h, o)


# speedup vs baseline: 34.5439x; 1.0321x over previous
import functools

import jax
import jax.numpy as jnp
from jax import lax
from jax.experimental import pallas as pl
from jax.experimental.pallas import tpu as pltpu
from jax.experimental.pallas import tpu_sc as plsc

N = 10000
NP2 = N // 2
F0 = 128
H = 64
E = 320000
EPS = 1e-5
SLOPE = 0.01

NC = 2
NS = 16
NW = NC * NS
CHUNK = 125
NCH = E // (NW * CHUNK)
RPT = N // NS
DEG_R = 640
BR = 1000
NBLK = NP2 // BR
BPW = 1024 // NW


def _sc_mesh():
    return plsc.VectorSubcoreMesh(
        core_axis_name="c", subcore_axis_name="s", num_cores=NC, num_subcores=NS
    )


_SC_PARAMS = pltpu.CompilerParams(use_tc_tiling_on_sc=False)


def _deg_body(dst_hbm, out_hbm, dstv, onesv, zv, deg_sp):
    c = lax.axis_index("c")
    s = lax.axis_index("s")
    w = c * NS + s
    for k in range(128 // 16):
        onesv[pl.ds(k * 16, 16)] = jnp.ones((16,), jnp.float32)

    def zbody(k, _):
        zv[pl.ds(k * 16, 16)] = jnp.zeros((16,), jnp.float32)
        return 0

    lax.fori_loop(0, DEG_R // 16, zbody, 0)
    pltpu.sync_copy(zv, deg_sp.at[pl.ds(s * DEG_R, DEG_R)])
    pltpu.sync_copy(dst_hbm.at[w], dstv)
    plsc.subcore_barrier()

    def body(j, _):
        pltpu.sync_copy(
            onesv.at[pl.ds(0, CHUNK)], deg_sp.at[dstv.at[j]], add=True
        )
        return 0

    lax.fori_loop(0, NCH, body, 0)
    plsc.subcore_barrier()
    pltpu.sync_copy(deg_sp.at[pl.ds(s * DEG_R, DEG_R)], zv)
    pltpu.sync_copy(zv, out_hbm.at[c, s])


def _deg_call(dstp):
    f = functools.partial(
        pl.kernel,
        out_type=jax.ShapeDtypeStruct((NC, NS, DEG_R), jnp.float32),
        mesh=_sc_mesh(),
        scratch_types=[
            pltpu.VMEM((NCH, CHUNK), jnp.int32),
            pltpu.VMEM((128,), jnp.float32),
            pltpu.VMEM((DEG_R,), jnp.float32),
            pltpu.VMEM_SHARED((NS * DEG_R,), jnp.float32),
        ],
        compiler_params=_SC_PARAMS,
        name="gcn_deg",
    )(_deg_body)
    return f(dstp)


def _mp_body(y_hbm, src_hbm, dst_hbm, zero_hbm, out_hbm,
             srcv, dstv, rows_a, rows_b, acc_sp,
             sem_ga, sem_gb, sem_sa, sem_sb):
    c = lax.axis_index("c")
    s = lax.axis_index("s")
    w = c * NS + s
    r0 = s * RPT
    pltpu.sync_copy(zero_hbm.at[pl.ds(r0, RPT)], acc_sp.at[pl.ds(r0, RPT)])
    pltpu.sync_copy(src_hbm.at[w], srcv)
    pltpu.sync_copy(dst_hbm.at[w], dstv)
    plsc.subcore_barrier()

    def gather(j, rows, sem):
        return pltpu.async_copy(y_hbm.at[srcv.at[j]], rows, sem)

    def gwait(j, rows, sem):
        pltpu.make_async_copy(y_hbm.at[srcv.at[j]], rows, sem).wait()

    def scatter(j, rows, sem):
        return pltpu.async_copy(rows, acc_sp.at[dstv.at[j]], sem, add=True)

    def swait(j, rows, sem):
        pltpu.make_async_copy(rows, acc_sp.at[dstv.at[j]], sem).wait()

    gather(0, rows_a, sem_ga)
    gather(1, rows_b, sem_gb)

    def body(k, _):
        j0 = 2 * k
        gwait(j0, rows_a, sem_ga)
        scatter(j0, rows_a, sem_sa)
        gwait(j0 + 1, rows_b, sem_gb)
        scatter(j0 + 1, rows_b, sem_sb)
        swait(j0, rows_a, sem_sa)
        gather(j0 + 2, rows_a, sem_ga)
        swait(j0 + 1, rows_b, sem_sb)
        gather(j0 + 3, rows_b, sem_gb)
        return 0

    lax.fori_loop(0, NCH // 2 - 1, body, 0)
    jl = NCH - 2
    gwait(jl, rows_a, sem_ga)
    scatter(jl, rows_a, sem_sa)
    gwait(jl + 1, rows_b, sem_gb)
    scatter(jl + 1, rows_b, sem_sb)
    swait(jl, rows_a, sem_sa)
    swait(jl + 1, rows_b, sem_sb)

    plsc.subcore_barrier()
    pltpu.sync_copy(acc_sp.at[pl.ds(r0, RPT)], out_hbm.at[c, pl.ds(r0, RPT)])


def _mp_call(y, srcp, dstp, zeros2d):
    f = functools.partial(
        pl.kernel,
        out_type=jax.ShapeDtypeStruct((NC, N, H), jnp.float32),
        mesh=_sc_mesh(),
        scratch_types=[
            pltpu.VMEM((NCH, CHUNK), jnp.int32),
            pltpu.VMEM((NCH, CHUNK), jnp.int32),
            pltpu.VMEM((CHUNK, H), jnp.float32),
            pltpu.VMEM((CHUNK, H), jnp.float32),
            pltpu.VMEM_SHARED((N, H), jnp.float32),
            pltpu.SemaphoreType.DMA,
            pltpu.SemaphoreType.DMA,
            pltpu.SemaphoreType.DMA,
            pltpu.SemaphoreType.DMA,
        ],
        compiler_params=_SC_PARAMS,
        name="gcn_msg_pass",
    )(_mp_body)
    return f(y, srcp, dstp, zeros2d)


def _sel_body(t_hbm, idx_hbm, out_hbm, idxv, rows, sem):
    c = lax.axis_index("c")
    s = lax.axis_index("s")
    w = c * NS + s
    base = w * BPW
    pltpu.sync_copy(idx_hbm.at[pl.ds(base, BPW)], idxv)
    pltpu.async_copy(t_hbm.at[idxv], rows, sem).wait()
    pltpu.sync_copy(rows, out_hbm.at[pl.ds(base, BPW)])


def _sel_call(t, idx):
    f = functools.partial(
        pl.kernel,
        out_type=jax.ShapeDtypeStruct((1024, H), jnp.float32),
        mesh=_sc_mesh(),
        scratch_types=[
            pltpu.VMEM((BPW,), jnp.int32),
            pltpu.VMEM((BPW, H), jnp.float32),
            pltpu.SemaphoreType.DMA,
        ],
        compiler_params=_SC_PARAMS,
        name="gcn_select",
    )(_sel_body)
    return f(t, idx)


def _ds_of(dd_ref):
    deg = dd_ref[0] + dd_ref[1] + 1.0
    dinv = lax.rsqrt(deg)
    lane = lax.broadcasted_iota(jnp.int32, (1, F0), 1)
    return jnp.where(lane < H, dinv[:, 0:1], dinv[:, 1:2])


def _ab_body(x_ref, w2_ref, dd_ref, y0_ref):
    xw = jnp.dot(x_ref[...], w2_ref[...], preferred_element_type=jnp.float32)
    y0_ref[...] = xw * _ds_of(dd_ref)


def _ab_call(x2, W02, dd):
    return pl.pallas_call(
        _ab_body,
        grid=(NBLK,),
        in_specs=[
            pl.BlockSpec((BR, 2 * F0), lambda i: (i, 0)),
            pl.BlockSpec((2 * F0, F0), lambda i: (0, 0)),
            pl.BlockSpec((2, BR, 2), lambda i: (0, i, 0)),
        ],
        out_specs=pl.BlockSpec((BR, F0), lambda i: (i, 0)),
        out_shape=jax.ShapeDtypeStruct((NP2, F0), jnp.float32),
        name="gcn_xw_dinv",
    )(x2, W02, dd)


def _post_body(s0_ref, s1_ref, y_ref, dd_ref, b2_ref, t_ref, st_ref):
    i = pl.program_id(0)
    tot = (s0_ref[...] + s1_ref[...] + y_ref[...]) * _ds_of(dd_ref) + b2_ref[...]
    t = jnp.where(tot > 0, tot, SLOPE * tot)
    t_ref[...] = t
    s1 = jnp.sum(t, axis=0, keepdims=True)
    s2 = jnp.sum(t * t, axis=0, keepdims=True)
    st = jnp.concatenate([s1, s2, jnp.zeros((6, F0), jnp.float32)], axis=0)

    @pl.when(i == 0)
    def _():
        st_ref[...] = st

    @pl.when(i != 0)
    def _():
        st_ref[...] += st


def _post_call(scat0, scat1, y, dd, b2):
    return pl.pallas_call(
        _post_body,
        grid=(NBLK,),
        in_specs=[
            pl.BlockSpec((BR, F0), lambda i: (i, 0)),
            pl.BlockSpec((BR, F0), lambda i: (i, 0)),
            pl.BlockSpec((BR, F0), lambda i: (i, 0)),
            pl.BlockSpec((2, BR, 2), lambda i: (0, i, 0)),
            pl.BlockSpec((1, F0), lambda i: (0, 0)),
        ],
        out_specs=[
            pl.BlockSpec((BR, F0), lambda i: (i, 0)),
            pl.BlockSpec((8, F0), lambda i: (0, 0)),
        ],
        out_shape=[
            jax.ShapeDtypeStruct((NP2, F0), jnp.float32),
            jax.ShapeDtypeStruct((8, F0), jnp.float32),
        ],
        name="gcn_post_stats",
    )(scat0, scat1, y, dd, b2)


def _bn_affine(st_ref, g2_ref, be2_ref):
    s1 = st_ref[0:1, :]
    s2 = st_ref[1:2, :]
    m64 = (s1[:, :H] + s1[:, H:]) / float(N)
    q64 = (s2[:, :H] + s2[:, H:]) / float(N)
    var64 = q64 - m64 * m64
    mean2 = jnp.concatenate([m64, m64], axis=1)
    var2 = jnp.concatenate([var64, var64], axis=1)
    alpha2 = g2_ref[...] * lax.rsqrt(var2 + EPS)
    return mean2, alpha2, be2_ref[...]


def _bn_mm_body(t_ref, st_ref, g2_ref, be2_ref, w12_ref, dd_ref, y1_ref):
    mean2, alpha2, be2 = _bn_affine(st_ref, g2_ref, be2_ref)
    h = (t_ref[...] - mean2) * alpha2 + be2
    y1 = jnp.dot(h, w12_ref[...], preferred_element_type=jnp.float32)
    y1_ref[...] = y1 * _ds_of(dd_ref)


def _bn_mm_call(t0, st0, g2, be2, W12, dd):
    return pl.pallas_call(
        _bn_mm_body,
        grid=(NBLK,),
        in_specs=[
            pl.BlockSpec((BR, F0), lambda i: (i, 0)),
            pl.BlockSpec((8, F0), lambda i: (0, 0)),
            pl.BlockSpec((1, F0), lambda i: (0, 0)),
            pl.BlockSpec((1, F0), lambda i: (0, 0)),
            pl.BlockSpec((F0, F0), lambda i: (0, 0)),
            pl.BlockSpec((2, BR, 2), lambda i: (0, i, 0)),
        ],
        out_specs=pl.BlockSpec((BR, F0), lambda i: (i, 0)),
        out_shape=jax.ShapeDtypeStruct((NP2, F0), jnp.float32),
        name="gcn_bn_mm",
    )(t0, st0, g2, be2, W12, dd)


def _head_body(tsel_ref, st_ref, g2_ref, be2_ref, wm2_ref, bm2_ref,
               h_ref, o_ref):
    mean2, alpha2, be2 = _bn_affine(st_ref, g2_ref, be2_ref)
    h = (tsel_ref[...] - mean2) * alpha2 + be2
    h_ref[...] = h
    z = jnp.dot(h, wm2_ref[...], preferred_element_type=jnp.float32)
    o_ref[...] = jax.nn.sigmoid(z + bm2_ref[...])


def _head_call(tsel2, st1, g2, be2, Wm2, bm2):
    return pl.pallas_call(
        _head_body,
        grid=(1,),
        in_specs=[
            pl.BlockSpec((512, F0), lambda i: (0, 0)),
            pl.BlockSpec((8, F0), lambda i: (0, 0)),
            pl.BlockSpec((1, F0), lambda i: (0, 0)),
            pl.BlockSpec((1, F0), lambda i: (0, 0)),
            pl.BlockSpec((F0, F0), lambda i: (0, 0)),
            pl.BlockSpec((1, F0), lambda i: (0, 0)),
        ],
        out_specs=[
            pl.BlockSpec((512, F0), lambda i: (0, 0)),
            pl.BlockSpec((512, F0), lambda i: (0, 0)),
        ],
        out_shape=[
            jax.ShapeDtypeStruct((512, F0), jnp.float32),
            jax.ShapeDtypeStruct((512, F0), jnp.float32),
        ],
        name="gcn_head",
    )(tsel2, st1, g2, be2, Wm2, bm2)


def _blockdiag(W):
    k, m = W.shape
    z = jnp.zeros((k, m), W.dtype)
    return jnp.concatenate(
        [jnp.concatenate([W, z], axis=1), jnp.concatenate([z, W], axis=1)],
        axis=0,
    )


def _dup(v):
    return jnp.concatenate([v, v]).reshape(1, 2 * v.shape[0])


def kernel(x, edge_index, idx, W0, b0, g0, be0, W1, b1, g1, be1, Wm, bm):
    x2 = x.reshape(NP2, 2 * F0)
    srcp = edge_index[0].reshape(NW, NCH, CHUNK)
    dstp = edge_index[1].reshape(NW, NCH, CHUNK)
    zeros2d = jnp.zeros((N, H), jnp.float32)

    degp = _deg_call(dstp)
    dd = degp.reshape(NC, NS * DEG_R)[:, :N].reshape(NC, NP2, 2)

    W02 = _blockdiag(W0)
    y0 = _ab_call(x2, W02, dd)
    scat0 = _mp_call(y0.reshape(N, H), srcp, dstp, zeros2d)
    s2v = scat0.reshape(NC, NP2, F0)
    t0, st0 = _post_call(s2v[0], s2v[1], y0, dd, _dup(b0))
    y1 = _bn_mm_call(t0, st0, _dup(g0), _dup(be0), _blockdiag(W1), dd)
    scat1 = _mp_call(y1.reshape(N, H), srcp, dstp, zeros2d)
    s2v1 = scat1.reshape(NC, NP2, F0)
    t1, st1 = _post_call(s2v1[0], s2v1[1], y1, dd, _dup(b1))

    tsel = _sel_call(t1.reshape(N, H), idx)
    nm = Wm.shape[1]
    Wm2 = jnp.pad(_blockdiag(Wm), ((0, 0), (0, F0 - 2 * nm)))
    bm2 = jnp.pad(jnp.concatenate([bm, bm]), (0, F0 - 2 * nm)).reshape(1, F0)
    h2, o2 = _head_call(tsel.reshape(512, F0), st1, _dup(g1), _dup(be1),
                        Wm2, bm2)
    h = h2.reshape(1024, H)
    o = o2[:, : 2 * nm].reshape(1024, nm)
    return (h, o)
